# baseline (speedup 1.0000x reference)
import functools

import jax
import jax.numpy as jnp
from jax import lax
from jax.experimental import pallas as pl
from jax.experimental.pallas import tpu as pltpu
from jax.experimental.pallas import tpu_sc as plsc

N = 10000
NPAD = 10240
F_IN = 256
F_H = 128
F_HW = F_H // 2
F_OUT = 64
B = 4
E = 160000

NC = 2
NS = 16
LANES = 16
KGRP = F_HW // LANES

E_PER_TILE = E // NS
CHUNK = 80
NCHUNK = E_PER_TILE // CHUNK
SB = 5
NSB = NCHUNK // SB
NBUF = 3
RING_T = (NCHUNK + NBUF - 1) // NBUF
ROWS_PER_TILE = NPAD // NS


def _fc1_body(x_ref, w_ref, b_ref, o_ref):
    y = jnp.dot(x_ref[...], w_ref[...], preferred_element_type=jnp.float32)
    o_ref[...] = jnp.maximum(y + b_ref[...], 0.0).astype(jnp.bfloat16)


def _fc1(features, W1, b1):
    return pl.pallas_call(
        _fc1_body,
        grid=(10,),
        in_specs=[
            pl.BlockSpec((1000, F_IN), lambda i: (i, 0)),
            pl.BlockSpec((F_IN, F_H), lambda i: (0, 0)),
            pl.BlockSpec((1, F_H), lambda i: (0, 0)),
        ],
        out_specs=pl.BlockSpec((1000, F_H), lambda i: (i, 0)),
        out_shape=jax.ShapeDtypeStruct((N, F_H), jnp.bfloat16),
    )(features, W1, b1.reshape(1, F_H))


def _spmm_body(h_hbm, src_hbm, dst_hbm, ev_hbm, zeros_hbm, out_hbm,
               src_sb, dst_sb, ev_sb, gbufs, sbufs, acc,
               gsems, ssems, isems):
    c = lax.axis_index("c")
    s = lax.axis_index("s")
    r0 = s * ROWS_PER_TILE

    def _gather(q, par, jj):
        pltpu.async_copy(h_hbm.at[src_sb.at[par, jj]], gbufs.at[q],
                         gsems.at[q])

    def _wait_gather(q):
        pltpu.make_async_copy(h_hbm.at[src_sb.at[0, 0]], gbufs.at[q],
                              gsems.at[q]).wait()

    def _scatter(q, par, jj):
        pltpu.async_copy(sbufs.at[q], acc.at[dst_sb.at[par, jj]],
                         ssems.at[q], add=True)

    def _wait_scatter(q):
        pltpu.make_async_copy(sbufs.at[q], acc.at[dst_sb.at[0, 0]],
                              ssems.at[q]).wait()

    def _load_sb(ebase, u, par, sem):
        pltpu.async_copy(src_hbm.at[ebase, u], src_sb.at[par], sem)
        pltpu.async_copy(dst_hbm.at[ebase, u], dst_sb.at[par], sem)
        pltpu.async_copy(ev_hbm.at[ebase, u], ev_sb.at[par], sem)

    def _wait_sb(ebase, par, sem):
        pltpu.make_async_copy(src_hbm.at[ebase, 0], src_sb.at[par], sem).wait()
        pltpu.make_async_copy(dst_hbm.at[ebase, 0], dst_sb.at[par], sem).wait()
        pltpu.make_async_copy(ev_hbm.at[ebase, 0], ev_sb.at[par], sem).wait()

    def _scale(q, par, jj):
        def _grp(g, _):
            ev16 = ev_sb[par, jj, pl.ds(g * LANES, LANES)]
            for i in range(LANES):
                v = jnp.full((LANES,), ev16[i], jnp.float32)
                e = g * LANES + i
                for k in range(KGRP):
                    w = gbufs[q, e, pl.ds(k * LANES, LANES)]
                    a = lax.bitcast_convert_type(
                        lax.shift_left(w, 16), jnp.float32)
                    b = lax.bitcast_convert_type(
                        jnp.bitwise_and(w, jnp.int32(-65536)), jnp.float32)
                    sbufs[q, e, pl.ds(2 * k * LANES, LANES)] = a * v
                    sbufs[q, e, pl.ds((2 * k + 1) * LANES, LANES)] = b * v
            return 0
        lax.fori_loop(0, CHUNK // LANES, _grp, 0)

    def _batch(bi, _):
        b = c * (B // NC) + bi
        ebase = b * NS + s

        pltpu.sync_copy(zeros_hbm, acc.at[pl.ds(r0, ROWS_PER_TILE), :])
        plsc.subcore_barrier()

        pltpu.sync_copy(src_hbm.at[ebase, 0], src_sb.at[0])
        pltpu.sync_copy(dst_hbm.at[ebase, 0], dst_sb.at[0])
        pltpu.sync_copy(ev_hbm.at[ebase, 0], ev_sb.at[0])
        _load_sb(ebase, 1, 1, isems.at[1])

        _gather(0, 0, 0)
        _gather(1, 0, 1)

        def _ring(t, _):
            for p in range(NBUF):
                j = t * NBUF + p

                @pl.when(j < NCHUNK)
                def _():
                    u = j // SB
                    jj = j - u * SB
                    par = lax.rem(u, 2)
                    _wait_gather(p)

                    @pl.when(j >= NBUF)
                    def _():
                        _wait_scatter(p)
                    _scale(p, par, jj)
                    _scatter(p, par, jj)

                    jn = j + 2
                    qn = (p + 2) % NBUF

                    @pl.when(jn < NCHUNK)
                    def _():
                        un = jn // SB
                        jjn = jn - un * SB
                        parn = lax.rem(un, 2)

                        @pl.when(jjn == 0)
                        def _():
                            _wait_sb(ebase, parn, isems.at[parn])
                        _gather(qn, parn, jjn)

                    up = u + 1

                    @pl.when((jj == 1) & (up >= 2) & (up < NSB))
                    def _():
                        _load_sb(ebase, up, lax.rem(up, 2),
                                 isems.at[lax.rem(up, 2)])
            return 0
        lax.fori_loop(0, RING_T, _ring, 0)

        for q in range(NBUF):
            _wait_scatter(q)
        plsc.subcore_barrier()

        pltpu.sync_copy(acc.at[pl.ds(r0, ROWS_PER_TILE), :],
                        out_hbm.at[b, pl.ds(r0, ROWS_PER_TILE), :])
        plsc.subcore_barrier()
        return 0

    lax.fori_loop(0, B // NC, _batch, 0)


def _spmm(h_packed, src4, dst4, ev4, zeros):
    mesh = plsc.VectorSubcoreMesh(core_axis_name="c", subcore_axis_name="s",
                                  num_cores=NC, num_subcores=NS)
    fn = pl.kernel(
        _spmm_body,
        out_type=jax.ShapeDtypeStruct((B, NPAD, F_H), jnp.float32),
        mesh=mesh,
        scratch_types=[
            pltpu.VMEM((2, SB, CHUNK), jnp.int32),
            pltpu.VMEM((2, SB, CHUNK), jnp.int32),
            pltpu.VMEM((2, SB, CHUNK), jnp.float32),
            pltpu.VMEM((NBUF, CHUNK, F_HW), jnp.int32),
            pltpu.VMEM((NBUF, CHUNK, F_H), jnp.float32),
            pltpu.VMEM_SHARED((NPAD, F_H), jnp.float32),
            pltpu.SemaphoreType.DMA((NBUF,)),
            pltpu.SemaphoreType.DMA((NBUF,)),
            pltpu.SemaphoreType.DMA((2,)),
        ],
        compiler_params=pltpu.CompilerParams(use_tc_tiling_on_sc=False),
    )
    return fn(h_packed, src4, dst4, ev4, zeros)


def _head_body(x_ref, w_ref, b_ref, o_ref):
    x = jnp.maximum(x_ref[0], 0.0)
    y = jnp.dot(x, w_ref[...], preferred_element_type=jnp.float32)
    y = (y + b_ref[...])[:N]
    m = jnp.max(y, axis=0, keepdims=True)
    z = y - m
    lse = jnp.log(jnp.sum(jnp.exp(z), axis=0, keepdims=True))
    o_ref[0] = z - lse


def _head(batch_h_pad, W2, b2):
    return pl.pallas_call(
        _head_body,
        grid=(B,),
        in_specs=[
            pl.BlockSpec((1, NPAD, F_H), lambda b: (b, 0, 0)),
            pl.BlockSpec((F_H, F_OUT), lambda b: (0, 0)),
            pl.BlockSpec((1, F_OUT), lambda b: (0, 0)),
        ],
        out_specs=pl.BlockSpec((1, N, F_OUT), lambda b: (b, 0, 0)),
        out_shape=jax.ShapeDtypeStruct((B, N, F_OUT), jnp.float32),
    )(batch_h_pad, W2, b2.reshape(1, F_OUT))


def kernel(features, edge_index, edge_values, W1, b1, W2, b2):
    h16 = _fc1(features, W1, b1)
    hp = h16.reshape(N, KGRP, 2, LANES).transpose(0, 1, 3, 2)
    hi = lax.bitcast_convert_type(hp.reshape(N, F_HW, 2), jnp.int32)
    hi = jnp.pad(hi, ((0, NPAD - N), (0, 0)))
    src4 = edge_index[:, 1, :].reshape(B * NS, NSB, SB, CHUNK)
    dst4 = edge_index[:, 0, :].reshape(B * NS, NSB, SB, CHUNK)
    ev4 = edge_values.reshape(B * NS, NSB, SB, CHUNK)
    zeros = jnp.zeros((ROWS_PER_TILE, F_H), jnp.float32)
    batch_h = _spmm(hi, src4, dst4, ev4, zeros)
    return _head(batch_h, W2, b2)


# speedup vs baseline: 1.7606x; 1.7606x over previous
import functools

import jax
import jax.numpy as jnp
from jax import lax
from jax.experimental import pallas as pl
from jax.experimental.pallas import tpu as pltpu
from jax.experimental.pallas import tpu_sc as plsc

N = 10000
NPAD = 10240
F_IN = 256
F_H = 128
F_OUT = 64
B = 4
E = 160000

NC = 2
NS = 16
LANES = 16
KGRP = F_H // LANES

E_PER_TILE = E // NS
CHUNK = 80
NCHUNK = E_PER_TILE // CHUNK
SB = 5
NSB = NCHUNK // SB
NBUF = 4
RING_T = (NCHUNK + NBUF - 1) // NBUF
ROWS_PER_TILE = NPAD // NS


def _fc1_body(x_ref, w_ref, b_ref, o_ref):
    y = jnp.dot(x_ref[...], w_ref[...], preferred_element_type=jnp.float32)
    o_ref[...] = jnp.maximum(y + b_ref[...], 0.0)


def _fc1(features, W1, b1):
    return pl.pallas_call(
        _fc1_body,
        grid=(10,),
        in_specs=[
            pl.BlockSpec((1000, F_IN), lambda i: (i, 0)),
            pl.BlockSpec((F_IN, F_H), lambda i: (0, 0)),
            pl.BlockSpec((1, F_H), lambda i: (0, 0)),
        ],
        out_specs=pl.BlockSpec((1000, F_H), lambda i: (i, 0)),
        out_shape=jax.ShapeDtypeStruct((N, F_H), jnp.float32),
    )(features, W1, b1.reshape(1, F_H))


def _spmm_body(h_hbm, src_hbm, dst_hbm, ev_hbm, zeros_hbm, out_hbm,
               src_sb, dst_sb, ev_sb, bufs, acc, gsems, ssems, isems):
    c = lax.axis_index("c")
    s = lax.axis_index("s")
    r0 = s * ROWS_PER_TILE

    def _gather(j, q, par, jj):
        pltpu.async_copy(h_hbm.at[src_sb.at[par, jj]], bufs.at[q],
                         gsems.at[q])

    def _wait_gather(q):
        pltpu.make_async_copy(h_hbm.at[src_sb.at[0, 0]], bufs.at[q],
                              gsems.at[q]).wait()

    def _scatter(j, q, par, jj):
        pltpu.async_copy(bufs.at[q], acc.at[dst_sb.at[par, jj]], ssems.at[q],
                         add=True)

    def _wait_scatter(q):
        pltpu.make_async_copy(bufs.at[q], acc.at[dst_sb.at[0, 0]],
                              ssems.at[q]).wait()

    def _load_sb(ebase, u, par, sem):
        pltpu.async_copy(src_hbm.at[ebase, u], src_sb.at[par], sem)
        pltpu.async_copy(dst_hbm.at[ebase, u], dst_sb.at[par], sem)
        pltpu.async_copy(ev_hbm.at[ebase, u], ev_sb.at[par], sem)

    def _wait_sb(ebase, par, sem):
        pltpu.make_async_copy(src_hbm.at[ebase, 0], src_sb.at[par], sem).wait()
        pltpu.make_async_copy(dst_hbm.at[ebase, 0], dst_sb.at[par], sem).wait()
        pltpu.make_async_copy(ev_hbm.at[ebase, 0], ev_sb.at[par], sem).wait()

    def _scale(q, par, jj):
        def _grp(g, _):
            ev16 = ev_sb[par, jj, pl.ds(g * LANES, LANES)]
            for i in range(LANES):
                v = jnp.full((LANES,), ev16[i], jnp.float32)
                e = g * LANES + i
                for k in range(KGRP):
                    sl = pl.ds(k * LANES, LANES)
                    bufs[q, e, sl] = bufs[q, e, sl] * v
            return 0
        lax.fori_loop(0, CHUNK // LANES, _grp, 0)

    def _batch(bi, _):
        b = c * (B // NC) + bi
        ebase = b * NS + s

        pltpu.sync_copy(zeros_hbm, acc.at[pl.ds(r0, ROWS_PER_TILE), :])
        plsc.subcore_barrier()

        pltpu.sync_copy(src_hbm.at[ebase, 0], src_sb.at[0])
        pltpu.sync_copy(dst_hbm.at[ebase, 0], dst_sb.at[0])
        pltpu.sync_copy(ev_hbm.at[ebase, 0], ev_sb.at[0])
        _load_sb(ebase, 1, 1, isems.at[1])

        for jp in range(NBUF - 1):
            _gather(jp, jp, 0, jp)

        def _ring(t, _):
            for p in range(NBUF):
                j = t * NBUF + p

                @pl.when(j < NCHUNK)
                def _():
                    u = j // SB
                    jj = j - u * SB
                    par = lax.rem(u, 2)
                    _wait_gather(p)
                    _scale(p, par, jj)
                    _scatter(j, p, par, jj)

                    jn = j + NBUF - 1
                    qn = (p + NBUF - 1) % NBUF

                    @pl.when(jn < NCHUNK)
                    def _():
                        un = jn // SB
                        jjn = jn - un * SB
                        parn = lax.rem(un, 2)

                        @pl.when(jjn == 0)
                        def _():
                            _wait_sb(ebase, parn, isems.at[parn])

                        @pl.when(jn >= NBUF)
                        def _():
                            _wait_scatter(qn)
                        _gather(jn, qn, parn, jjn)

                    up = u + 1

                    @pl.when((jj == 1) & (up >= 2) & (up < NSB))
                    def _():
                        _load_sb(ebase, up, lax.rem(up, 2), isems.at[lax.rem(up, 2)])
            return 0
        lax.fori_loop(0, RING_T, _ring, 0)

        for q in range(NBUF):
            _wait_scatter(q)
        plsc.subcore_barrier()

        pltpu.sync_copy(acc.at[pl.ds(r0, ROWS_PER_TILE), :],
                        out_hbm.at[b, pl.ds(r0, ROWS_PER_TILE), :])
        plsc.subcore_barrier()
        return 0

    lax.fori_loop(0, B // NC, _batch, 0)


def _spmm(h, src4, dst4, ev4, zeros):
    mesh = plsc.VectorSubcoreMesh(core_axis_name="c", subcore_axis_name="s",
                                  num_cores=NC, num_subcores=NS)
    fn = pl.kernel(
        _spmm_body,
        out_type=jax.ShapeDtypeStruct((B, NPAD, F_H), jnp.float32),
        mesh=mesh,
        scratch_types=[
            pltpu.VMEM((2, SB, CHUNK), jnp.int32),
            pltpu.VMEM((2, SB, CHUNK), jnp.int32),
            pltpu.VMEM((2, SB, CHUNK), jnp.float32),
            pltpu.VMEM((NBUF, CHUNK, F_H), jnp.float32),
            pltpu.VMEM_SHARED((NPAD, F_H), jnp.float32),
            pltpu.SemaphoreType.DMA((NBUF,)),
            pltpu.SemaphoreType.DMA((NBUF,)),
            pltpu.SemaphoreType.DMA((2,)),
        ],
    )
    return fn(h, src4, dst4, ev4, zeros)


def _head_body(x_ref, w_ref, b_ref, o_ref):
    x = jnp.maximum(x_ref[0], 0.0)
    y = jnp.dot(x, w_ref[...], preferred_element_type=jnp.float32)
    y = (y + b_ref[...])[:N]
    m = jnp.max(y, axis=0, keepdims=True)
    z = y - m
    lse = jnp.log(jnp.sum(jnp.exp(z), axis=0, keepdims=True))
    o_ref[0] = z - lse


def _head(batch_h_pad, W2, b2):
    return pl.pallas_call(
        _head_body,
        grid=(B,),
        in_specs=[
            pl.BlockSpec((1, NPAD, F_H), lambda b: (b, 0, 0)),
            pl.BlockSpec((F_H, F_OUT), lambda b: (0, 0)),
            pl.BlockSpec((1, F_OUT), lambda b: (0, 0)),
        ],
        out_specs=pl.BlockSpec((1, N, F_OUT), lambda b: (b, 0, 0)),
        out_shape=jax.ShapeDtypeStruct((B, N, F_OUT), jnp.float32),
    )(batch_h_pad, W2, b2.reshape(1, F_OUT))


def kernel(features, edge_index, edge_values, W1, b1, W2, b2):
    h = _fc1(features, W1, b1)
    src4 = edge_index[:, 1, :].reshape(B * NS, NSB, SB, CHUNK)
    dst4 = edge_index[:, 0, :].reshape(B * NS, NSB, SB, CHUNK)
    ev4 = edge_values.reshape(B * NS, NSB, SB, CHUNK)
    zeros = jnp.zeros((ROWS_PER_TILE, F_H), jnp.float32)
    batch_h = _spmm(h, src4, dst4, ev4, zeros)
    return _head(batch_h, W2, b2)
